# baseline (speedup 1.0000x reference)
import functools
import jax
import jax.numpy as jnp
from jax import lax
from jax.experimental import pallas as pl
from jax.experimental.pallas import tpu as pltpu
from jax.experimental.pallas import tpu_sc as plsc

VOCAB = 100004
EMBED_DIM = 128
BATCH = 4096
HIST_LEN = 200

_TOTAL = BATCH * HIST_LEN
_IDX_COLS = 128
_IDX_ROWS = _TOTAL // _IDX_COLS
_NW = 32
_ROWS_PER_W = _IDX_ROWS // _NW


_NBUF = 4


def _gather_body(idx_hbm, table_hbm, out_hbm, idx_v,
                 rows0, rows1, rows2, rows3,
                 sg0, sg1, sg2, sg3, ss0, ss1, ss2, ss3):
    wid = lax.axis_index("s") * 2 + lax.axis_index("c")
    row_base = wid * _ROWS_PER_W

    pltpu.sync_copy(idx_hbm.at[pl.ds(row_base, _ROWS_PER_W)], idx_v)

    rows = (rows0, rows1, rows2, rows3)
    sg = (sg0, sg1, sg2, sg3)
    ss = (ss0, ss1, ss2, ss3)

    def gather_start(g, b):
        pltpu.async_copy(table_hbm.at[idx_v.at[g]], rows[b], sg[b])

    def wait_gather(b):
        pltpu.make_async_copy(table_hbm.at[idx_v.at[0]], rows[b], sg[b]).wait()

    def store_start(g, b):
        pltpu.async_copy(
            rows[b], out_hbm.at[pl.ds((row_base + g) * _IDX_COLS, _IDX_COLS)],
            ss[b],
        )

    def wait_store(b):
        pltpu.make_async_copy(
            rows[b], out_hbm.at[pl.ds(row_base * _IDX_COLS, _IDX_COLS)], ss[b]
        ).wait()

    gather_start(0, 0)
    gather_start(1, 1)
    wait_gather(0)
    store_start(0, 0)
    gather_start(2, 2)
    wait_gather(1)
    store_start(1, 1)
    gather_start(3, 3)

    @pl.loop(2, _ROWS_PER_W - 2, step=_NBUF)
    def _(g0):
        for b in range(_NBUF):
            g = g0 + b
            buf = (2 + b) % _NBUF
            wait_gather(buf)
            store_start(g, buf)
            wait_store(b)
            gather_start(g + 2, b)

    wait_gather(2)
    store_start(_ROWS_PER_W - 2, 2)
    wait_gather(3)
    store_start(_ROWS_PER_W - 1, 3)
    for b in range(_NBUF):
        wait_store(b)


def kernel(input_indices, embedding_matrix):
    idx2d = input_indices.reshape(_IDX_ROWS, _IDX_COLS)

    mesh = plsc.VectorSubcoreMesh(core_axis_name="c", subcore_axis_name="s")
    out_flat = pl.kernel(
        _gather_body,
        mesh=mesh,
        out_type=jax.ShapeDtypeStruct((_TOTAL, EMBED_DIM), jnp.float32),
        scratch_types=(
            [pltpu.VMEM((_ROWS_PER_W, _IDX_COLS), jnp.int32)]
            + [pltpu.VMEM((_IDX_COLS, EMBED_DIM), jnp.float32)] * _NBUF
            + [pltpu.SemaphoreType.DMA] * (2 * _NBUF)
        ),
    )(idx2d, embedding_matrix)

    return out_flat.reshape(BATCH, HIST_LEN, EMBED_DIM)
